# baseline (speedup 1.0000x reference)
import functools

import jax
import jax.numpy as jnp
from jax import lax
from jax.experimental import pallas as pl
from jax.experimental.pallas import tpu as pltpu
from jax.experimental.pallas import tpu_sc as plsc

NF = 128
NS = 100000
NC = 500
CPAD = 512
B = 128
TEMP = 0.05
NTILES = 32
CHUNK = 128
NCHUNKS = 24
MAIN_ROWS = NTILES * NCHUNKS * CHUNK
TAIL_A = 80
TAIL_N = 21
TAIL_B = 16

_mesh = plsc.VectorSubcoreMesh(core_axis_name="c", subcore_axis_name="s")


@functools.partial(
    pl.kernel,
    out_type=[
        jax.ShapeDtypeStruct((NTILES, CPAD, NF), jnp.float32),
        jax.ShapeDtypeStruct((NTILES, CPAD // 8, NF), jnp.float32),
    ],
    mesh=_mesh,
    compiler_params=pltpu.CompilerParams(needs_layout_passes=False),
    scratch_types=[
        pltpu.VMEM((CPAD, NF), jnp.float32),
        pltpu.VMEM((CPAD // 8, NF), jnp.float32),
        pltpu.VMEM((2, CHUNK, NF), jnp.float32),
        pltpu.VMEM((2, CHUNK), jnp.int32),
        pltpu.VMEM((2, CHUNK), jnp.int32),
        pltpu.VMEM((CHUNK + 16,), jnp.int32),
        pltpu.VMEM((CHUNK + 16,), jnp.int32),
        pltpu.SemaphoreType.DMA,
        pltpu.SemaphoreType.DMA,
        pltpu.SemaphoreType.DMA,
        pltpu.SemaphoreType.DMA,
        pltpu.SemaphoreType.DMA,
        pltpu.SemaphoreType.DMA,
    ],
)
def _sc_segment_sum(feat_hbm, lab_hbm, lab1_hbm, outg_hbm, outc_hbm,
                    acc, cnt, fbuf, lbuf, l1buf, wl_rid, wl_lab,
                    semf0, seml0, sem10, semf1, seml1, sem11):
    wid = lax.axis_index("s") * 2 + lax.axis_index("c")
    zero16 = jnp.zeros((16,), jnp.float32)
    sems = [(semf0, seml0, sem10), (semf1, seml1, sem11)]
    lane_iota = lax.iota(jnp.int32, 16)
    ones16 = jnp.ones((16,), jnp.float32)

    def issue(base, nrows, b):
        semf, seml, sem1 = sems[b]
        pltpu.async_copy(feat_hbm.at[pl.ds(base, nrows)],
                         fbuf.at[b, pl.ds(0, nrows)], semf)
        pltpu.async_copy(lab_hbm.at[pl.ds(base, nrows)],
                         lbuf.at[b, pl.ds(0, nrows)], seml)
        pltpu.async_copy(lab1_hbm.at[pl.ds(base, nrows)],
                         l1buf.at[b, pl.ds(0, nrows)], sem1)

    def wait_buf(nrows, b):
        semf, seml, sem1 = sems[b]
        pltpu.make_async_copy(feat_hbm.at[pl.ds(0, nrows)],
                              fbuf.at[b, pl.ds(0, nrows)], semf).wait()
        pltpu.make_async_copy(lab_hbm.at[pl.ds(0, nrows)],
                              lbuf.at[b, pl.ds(0, nrows)], seml).wait()
        pltpu.make_async_copy(lab1_hbm.at[pl.ds(0, nrows)],
                              l1buf.at[b, pl.ds(0, nrows)], sem1).wait()

    def process(nrows, b):
        labs, masks = [], []
        for j in range(nrows // 16):
            lab_vec = lbuf[b, pl.ds(j * 16, 16)]
            l1_vec = l1buf[b, pl.ds(j * 16, 16)]
            mask = l1_vec <= 0
            labs.append(lab_vec)
            masks.append(mask)
            labm = jnp.where(mask, lab_vec, CPAD - 1)
            crow = lax.shift_right_logical(labm, 3)
            ccol = jnp.bitwise_or(
                lax.shift_left(jnp.bitwise_and(labm, 7), 4), lane_iota)
            plsc.addupdate_scatter(cnt, [crow, ccol], ones16)
        ns = [plsc.all_reduce_population_count(m)[0] for m in masks]
        off = jnp.int32(0)
        for j in range(nrows // 16):
            plsc.store_compressed(wl_rid.at[pl.ds(off, 16)],
                                  j * 16 + lane_iota, mask=masks[j])
            plsc.store_compressed(wl_lab.at[pl.ds(off, 16)], labs[j],
                                  mask=masks[j])
            off = off + ns[j]
        wl_rid[pl.ds(off, 16)] = jnp.zeros((16,), jnp.int32)
        wl_lab[pl.ds(off, 16)] = jnp.full((16,), CPAD - 1, jnp.int32)

        ngrp = lax.shift_right_logical(off + 15, 4)

        @plsc.parallel_loop(0, ngrp, unroll=1)
        def _grp(j):
            r0 = j * 16
            rid_vec = wl_rid[pl.ds(r0, 16)]
            lab_vec = wl_lab[pl.ds(r0, 16)]

            def load_row(l):
                rid = rid_vec[l]
                return [fbuf[b, rid, pl.ds(k * 16, 16)]
                        for k in range(NF // 16)]

            vals = load_row(0)
            labp = lab_vec[0]
            for l in range(16):
                if l + 1 < 16:
                    nvals = load_row(l + 1)
                    nlabp = lab_vec[l + 1]
                for k in range(NF // 16):
                    plsc.addupdate(acc.at[labp, pl.ds(k * 16, 16)], vals[k])
                if l + 1 < 16:
                    vals, labp = nvals, nlabp

    def chunk_base(i):
        return (i * NTILES + wid) * CHUNK

    issue(chunk_base(0), CHUNK, 0)
    issue(chunk_base(1), CHUNK, 1)

    @pl.loop(0, CPAD)
    def _zero(r):
        for k in range(NF // 16):
            acc[r, pl.ds(k * 16, 16)] = zero16

    @pl.loop(0, CPAD // 8)
    def _zeroc(r):
        for k in range(NF // 16):
            cnt[r, pl.ds(k * 16, 16)] = zero16

    @pl.loop(0, NCHUNKS // 2)
    def _chunk_pair(it):
        c = it * 2
        wait_buf(CHUNK, 0)
        process(CHUNK, 0)

        @pl.when(c + 2 < NCHUNKS)
        def _issue0():
            issue(chunk_base(c + 2), CHUNK, 0)

        wait_buf(CHUNK, 1)
        process(CHUNK, 1)

        @pl.when(c + 3 < NCHUNKS)
        def _issue1():
            issue(chunk_base(c + 3), CHUNK, 1)

    tail_base = MAIN_ROWS + wid * TAIL_A

    @pl.when(wid < TAIL_N)
    def _tail_a():
        issue(tail_base, TAIL_A, 0)
        wait_buf(TAIL_A, 0)
        process(TAIL_A, 0)

    @pl.when(wid == TAIL_N)
    def _tail_b():
        issue(MAIN_ROWS + TAIL_N * TAIL_A, TAIL_B, 0)
        wait_buf(TAIL_B, 0)
        process(TAIL_B, 0)

    pltpu.async_copy(acc, outg_hbm.at[wid], semf0).wait()
    pltpu.async_copy(cnt, outc_hbm.at[wid], semf0).wait()


GSTEP = 8
NSTEP = NTILES // GSTEP


def _tc_finish_body(pg_ref, pc_ref, in_ref, tgt_ref, w_ref, out_ref,
                    gacc, cacc):
    i = pl.program_id(0)
    psum = jnp.sum(pg_ref[...], axis=0)
    csum = jnp.sum(pc_ref[...], axis=0)

    @pl.when(i == 0)
    def _init():
        gacc[...] = psum
        cacc[...] = csum

    @pl.when(i > 0)
    def _accum():
        gacc[...] += psum
        cacc[...] += csum

    @pl.when(i == NSTEP - 1)
    def _final():
        _tc_loss(gacc[...], cacc[...], in_ref, tgt_ref, w_ref, out_ref)


def _tc_loss(gsum, cnt2, in_ref, tgt_ref, w_ref, out_ref):
    rsel = lax.shift_right_logical(
        lax.broadcasted_iota(jnp.int32, (CPAD, CPAD // 8), 0), 3)
    csel = lax.broadcasted_iota(jnp.int32, (CPAD, CPAD // 8), 1)
    psel = (rsel == csel).astype(jnp.float32)
    dn0 = (((1,), (0,)), ((), ()))
    t = lax.dot_general(psel, cnt2, dn0,
                        precision=lax.Precision.HIGHEST)
    colg = lax.shift_right_logical(
        lax.broadcasted_iota(jnp.int32, (CPAD, NF), 1), 4)
    cmod = jnp.bitwise_and(lax.broadcasted_iota(jnp.int32, (CPAD, NF), 0), 7)
    wsel = (colg == cmod).astype(jnp.float32)
    nums = jnp.sum(t * wsel, axis=1, keepdims=True)
    denom = jnp.where(nums > 0, nums, 1.0)
    inv = 1.0 / denom

    g = gsum * inv
    dn = (((1,), (1,)), ((), ()))
    vec = lax.dot_general(in_ref[...], g, dn,
                          precision=lax.Precision.HIGHEST) * (1.0 / TEMP)

    ones_col = jnp.ones((B, 1), jnp.float32)
    nums_b = lax.dot_general(ones_col, nums, dn,
                             precision=lax.Precision.HIGHEST)
    col_id = lax.broadcasted_iota(jnp.int32, (B, CPAD), 1)
    m = jnp.logical_and(col_id < NC, nums_b > 0)
    mf = m.astype(jnp.float32)

    vecm = jnp.where(m, vec, 0.0)
    exps = jnp.exp(vecm) * mf
    sums = jnp.sum(exps, axis=1, keepdims=True) + 1e-6
    masked_sim = exps / sums
    log_probs = jnp.log(masked_sim + 1e-6)
    lossmat = -tgt_ref[...] * log_probs[:, :NC]
    rs = jnp.sum(lossmat, axis=1, keepdims=True)
    total = lax.dot_general(w_ref[...], rs, (((1,), (0,)), ((), ())),
                            precision=lax.Precision.HIGHEST)
    out_ref[...] = total * (1.0 / B)


_tc_finish = pl.pallas_call(
    _tc_finish_body,
    grid=(NSTEP,),
    in_specs=[
        pl.BlockSpec((GSTEP, CPAD, NF), lambda i: (i, 0, 0)),
        pl.BlockSpec((GSTEP, CPAD // 8, NF), lambda i: (i, 0, 0)),
        pl.BlockSpec((B, NF), lambda i: (0, 0)),
        pl.BlockSpec((B, NC), lambda i: (0, 0)),
        pl.BlockSpec((1, B), lambda i: (0, 0)),
    ],
    out_specs=pl.BlockSpec((1, 1), lambda i: (0, 0)),
    out_shape=jax.ShapeDtypeStruct((1, 1), jnp.float32),
    scratch_shapes=[
        pltpu.VMEM((CPAD, NF), jnp.float32),
        pltpu.VMEM((CPAD // 8, NF), jnp.float32),
    ],
)


@jax.jit
def kernel(inputs, indexes, targets, weight, features, labels, labels_1):
    del indexes
    partg, partc = _sc_segment_sum(features, labels.astype(jnp.int32),
                                   labels_1.astype(jnp.int32))
    out = _tc_finish(partg, partc, inputs, targets, weight.reshape(1, B))
    return out[0, 0]


# speedup vs baseline: 1.0101x; 1.0101x over previous
import functools

import jax
import jax.numpy as jnp
from jax import lax
from jax.experimental import pallas as pl
from jax.experimental.pallas import tpu as pltpu
from jax.experimental.pallas import tpu_sc as plsc

NF = 128
NS = 100000
NC = 500
CPAD = 512
B = 128
TEMP = 0.05
NTILES = 32
CHUNK = 128
NCHUNKS = 24
MAIN_ROWS = NTILES * NCHUNKS * CHUNK
TAIL_A = 80
TAIL_N = 21
TAIL_B = 16

_mesh = plsc.VectorSubcoreMesh(core_axis_name="c", subcore_axis_name="s")


@functools.partial(
    pl.kernel,
    out_type=[
        jax.ShapeDtypeStruct((NTILES, CPAD, NF), jnp.float32),
        jax.ShapeDtypeStruct((NTILES, CPAD // 8, NF), jnp.float32),
    ],
    mesh=_mesh,
    compiler_params=pltpu.CompilerParams(needs_layout_passes=False),
    scratch_types=[
        pltpu.VMEM((CPAD, NF), jnp.float32),
        pltpu.VMEM((CPAD // 8, NF), jnp.float32),
        pltpu.VMEM((2, CHUNK, NF), jnp.float32),
        pltpu.VMEM((2, CHUNK), jnp.int32),
        pltpu.VMEM((2, CHUNK), jnp.int32),
        pltpu.VMEM((CHUNK + 16,), jnp.int32),
        pltpu.VMEM((CHUNK + 16,), jnp.int32),
        pltpu.SemaphoreType.DMA,
        pltpu.SemaphoreType.DMA,
        pltpu.SemaphoreType.DMA,
        pltpu.SemaphoreType.DMA,
        pltpu.SemaphoreType.DMA,
        pltpu.SemaphoreType.DMA,
    ],
)
def _sc_segment_sum(feat_hbm, lab_hbm, lab1_hbm, outg_hbm, outc_hbm,
                    acc, cnt, fbuf, lbuf, l1buf, wl_rid, wl_lab,
                    semf0, seml0, sem10, semf1, seml1, sem11):
    wid = lax.axis_index("s") * 2 + lax.axis_index("c")
    zero16 = jnp.zeros((16,), jnp.float32)
    sems = [(semf0, seml0, sem10), (semf1, seml1, sem11)]
    lane_iota = lax.iota(jnp.int32, 16)
    ones16 = jnp.ones((16,), jnp.float32)

    def issue(base, nrows, b):
        semf, seml, sem1 = sems[b]
        pltpu.async_copy(feat_hbm.at[pl.ds(base, nrows)],
                         fbuf.at[b, pl.ds(0, nrows)], semf)
        pltpu.async_copy(lab_hbm.at[pl.ds(base, nrows)],
                         lbuf.at[b, pl.ds(0, nrows)], seml)
        pltpu.async_copy(lab1_hbm.at[pl.ds(base, nrows)],
                         l1buf.at[b, pl.ds(0, nrows)], sem1)

    def wait_buf(nrows, b):
        semf, seml, sem1 = sems[b]
        pltpu.make_async_copy(feat_hbm.at[pl.ds(0, nrows)],
                              fbuf.at[b, pl.ds(0, nrows)], semf).wait()
        pltpu.make_async_copy(lab_hbm.at[pl.ds(0, nrows)],
                              lbuf.at[b, pl.ds(0, nrows)], seml).wait()
        pltpu.make_async_copy(lab1_hbm.at[pl.ds(0, nrows)],
                              l1buf.at[b, pl.ds(0, nrows)], sem1).wait()

    def process(nrows, b):
        labs, masks = [], []
        for j in range(nrows // 16):
            lab_vec = lbuf[b, pl.ds(j * 16, 16)]
            l1_vec = l1buf[b, pl.ds(j * 16, 16)]
            mask = l1_vec <= 0
            labs.append(lab_vec)
            masks.append(mask)
            labm = jnp.where(mask, lab_vec, CPAD - 1)
            crow = lax.shift_right_logical(labm, 3)
            ccol = jnp.bitwise_or(
                lax.shift_left(jnp.bitwise_and(labm, 7), 4), lane_iota)
            plsc.addupdate_scatter(cnt, [crow, ccol], ones16)
        ns = [plsc.all_reduce_population_count(m)[0] for m in masks]
        off = jnp.int32(0)
        for j in range(nrows // 16):
            plsc.store_compressed(wl_rid.at[pl.ds(off, 16)],
                                  j * 16 + lane_iota, mask=masks[j])
            plsc.store_compressed(wl_lab.at[pl.ds(off, 16)], labs[j],
                                  mask=masks[j])
            off = off + ns[j]
        wl_rid[pl.ds(off, 16)] = jnp.zeros((16,), jnp.int32)
        wl_lab[pl.ds(off, 16)] = jnp.full((16,), CPAD - 1, jnp.int32)

        ngrp = lax.shift_right_logical(off + 15, 4)

        @plsc.parallel_loop(0, ngrp, unroll=1)
        def _grp(j):
            r0 = j * 16
            rid_vec = wl_rid[pl.ds(r0, 16)]
            lab_vec = wl_lab[pl.ds(r0, 16)]

            def load_row(l):
                rid = rid_vec[l]
                return [fbuf[b, rid, pl.ds(k * 16, 16)]
                        for k in range(NF // 16)]

            vals = load_row(0)
            labp = lab_vec[0]
            for l in range(16):
                if l + 1 < 16:
                    nvals = load_row(l + 1)
                    nlabp = lab_vec[l + 1]
                for k in range(NF // 16):
                    plsc.addupdate(acc.at[labp, pl.ds(k * 16, 16)], vals[k])
                if l + 1 < 16:
                    vals, labp = nvals, nlabp

    def chunk_base(i):
        return (i * NTILES + wid) * CHUNK

    issue(chunk_base(0), CHUNK, 0)
    issue(chunk_base(1), CHUNK, 1)

    @pl.loop(0, CPAD)
    def _zero(r):
        for k in range(NF // 16):
            acc[r, pl.ds(k * 16, 16)] = zero16

    @pl.loop(0, CPAD // 8)
    def _zeroc(r):
        for k in range(NF // 16):
            cnt[r, pl.ds(k * 16, 16)] = zero16

    @pl.loop(0, NCHUNKS // 2)
    def _chunk_pair(it):
        c = it * 2
        wait_buf(CHUNK, 0)
        process(CHUNK, 0)

        @pl.when(c + 2 < NCHUNKS)
        def _issue0():
            issue(chunk_base(c + 2), CHUNK, 0)

        wait_buf(CHUNK, 1)
        process(CHUNK, 1)

        @pl.when(c + 3 < NCHUNKS)
        def _issue1():
            issue(chunk_base(c + 3), CHUNK, 1)

    tail_base = MAIN_ROWS + wid * TAIL_A

    @pl.when(wid < TAIL_N)
    def _tail_a():
        issue(tail_base, TAIL_A, 0)
        wait_buf(TAIL_A, 0)
        process(TAIL_A, 0)

    @pl.when(wid == TAIL_N)
    def _tail_b():
        issue(MAIN_ROWS + TAIL_N * TAIL_A, TAIL_B, 0)
        wait_buf(TAIL_B, 0)
        process(TAIL_B, 0)

    pltpu.async_copy(acc, outg_hbm.at[wid], semf0).wait()
    pltpu.async_copy(cnt, outc_hbm.at[wid], semf0).wait()


def _tc_finish_body(pg_ref, pc_ref, in_ref, tgt_ref, w_ref, out_ref):
    cnt2 = jnp.sum(pc_ref[...], axis=0)
    rsel = lax.shift_right_logical(
        lax.broadcasted_iota(jnp.int32, (CPAD, CPAD // 8), 0), 3)
    csel = lax.broadcasted_iota(jnp.int32, (CPAD, CPAD // 8), 1)
    psel = (rsel == csel).astype(jnp.float32)
    dn0 = (((1,), (0,)), ((), ()))
    t = lax.dot_general(psel, cnt2, dn0,
                        precision=lax.Precision.HIGHEST)
    colg = lax.shift_right_logical(
        lax.broadcasted_iota(jnp.int32, (CPAD, NF), 1), 4)
    cmod = jnp.bitwise_and(lax.broadcasted_iota(jnp.int32, (CPAD, NF), 0), 7)
    wsel = (colg == cmod).astype(jnp.float32)
    nums = jnp.sum(t * wsel, axis=1, keepdims=True)
    denom = jnp.where(nums > 0, nums, 1.0)
    inv = 1.0 / denom

    g = jnp.sum(pg_ref[...], axis=0) * inv
    dn = (((1,), (1,)), ((), ()))
    vec = lax.dot_general(in_ref[...], g, dn,
                          precision=lax.Precision.HIGHEST) * (1.0 / TEMP)

    ones_col = jnp.ones((B, 1), jnp.float32)
    nums_b = lax.dot_general(ones_col, nums, dn,
                             precision=lax.Precision.HIGHEST)
    col_id = lax.broadcasted_iota(jnp.int32, (B, CPAD), 1)
    m = jnp.logical_and(col_id < NC, nums_b > 0)
    mf = m.astype(jnp.float32)

    vecm = jnp.where(m, vec, 0.0)
    exps = jnp.exp(vecm) * mf
    sums = jnp.sum(exps, axis=1, keepdims=True) + 1e-6
    masked_sim = exps / sums
    log_probs = jnp.log(masked_sim + 1e-6)
    lossmat = -tgt_ref[...] * log_probs[:, :NC]
    rs = jnp.sum(lossmat, axis=1, keepdims=True)
    total = lax.dot_general(w_ref[...], rs, (((1,), (0,)), ((), ())),
                            precision=lax.Precision.HIGHEST)
    out_ref[...] = total * (1.0 / B)


_tc_finish = pl.pallas_call(
    _tc_finish_body,
    out_shape=jax.ShapeDtypeStruct((1, 1), jnp.float32),
)


@jax.jit
def kernel(inputs, indexes, targets, weight, features, labels, labels_1):
    del indexes
    partg, partc = _sc_segment_sum(features, labels.astype(jnp.int32),
                                   labels_1.astype(jnp.int32))
    out = _tc_finish(partg, partc, inputs, targets, weight.reshape(1, B))
    return out[0, 0]
